# baseline (speedup 1.0000x reference)
import functools

import jax
import jax.numpy as jnp
from jax import lax
from jax.experimental import pallas as pl
from jax.experimental.pallas import tpu as pltpu
from jax.experimental.pallas import tpu_sc as plsc

N = 10000
E = 320000
T = 5
D = 128
H = 128
O = 128

NC = 2
NS = 16
NW = NC * NS
NP = 10240
ROWS_PER_TILE = NP // NS
CHUNK = 120
PCHUNKS = 2688
SLAB = PCHUNKS // NW
NBUF = 3
IR = 2 * NBUF
EPAD = PCHUNKS * CHUNK - E

@functools.cache
def _mesh():
    return plsc.VectorSubcoreMesh(core_axis_name="c", subcore_axis_name="s",
                                  num_cores=NC, num_subcores=NS)


def _deg_body(eidx_hbm, zeros_hbm, deg_out, eidx_v, ones_v, deg_sh, ssem):
    c = lax.axis_index("c")
    s = lax.axis_index("s")
    wid = s * NC + c
    c0 = wid * SLAB
    pltpu.sync_copy(zeros_hbm, deg_sh.at[pl.ds(s * ROWS_PER_TILE, ROWS_PER_TILE)])
    pltpu.sync_copy(eidx_hbm.at[pl.ds(c0, SLAB)], eidx_v)
    for i in range(128 // 16):
        ones_v[pl.ds(i * 16, 16)] = jnp.full((16,), 1.0, jnp.float32)
    plsc.subcore_barrier()

    def fire(j, carry):
        pltpu.async_copy(ones_v.at[pl.ds(0, CHUNK)],
                         deg_sh.at[eidx_v.at[j, 1]], ssem, add=True)
        return carry

    lax.fori_loop(0, SLAB, fire, 0)

    def drain(j, carry):
        pltpu.make_async_copy(ones_v.at[pl.ds(0, CHUNK)],
                              deg_sh.at[eidx_v.at[0, 1]], ssem).wait()
        return carry

    lax.fori_loop(0, SLAB, drain, 0)

    plsc.subcore_barrier()
    pltpu.sync_copy(deg_sh.at[pl.ds(s * ROWS_PER_TILE, ROWS_PER_TILE)],
                    deg_out.at[c, pl.ds(s * ROWS_PER_TILE, ROWS_PER_TILE)])


@functools.cache
def _deg_call():
    return pl.kernel(
        _deg_body,
        out_type=jax.ShapeDtypeStruct((NC, NP), jnp.float32),
        mesh=_mesh(),
        scratch_types=[
            pltpu.VMEM((SLAB, 2, CHUNK), jnp.int32),
            pltpu.VMEM((128,), jnp.float32),
            pltpu.VMEM_SHARED((NP,), jnp.float32),
            pltpu.SemaphoreType.DMA,
        ],
    )


def _agg_body(eidx_hbm, y_hbm, zeros_hbm, agg_out,
              eidx_v, rows_v, agg_sh, *sems):
    gsem = sems[:NBUF]
    isem = sems[NBUF:]
    c = lax.axis_index("c")
    s = lax.axis_index("s")
    wid = s * NC + c
    c0 = wid * SLAB
    pltpu.sync_copy(zeros_hbm, agg_sh.at[pl.ds(s * ROWS_PER_TILE, ROWS_PER_TILE)])

    def idx_load(j, q):
        pltpu.async_copy(eidx_hbm.at[c0 + j], eidx_v.at[q], isem[q])

    def idx_wait(j, q):
        pltpu.make_async_copy(eidx_hbm.at[c0 + j], eidx_v.at[q], isem[q]).wait()

    def gather(j, b, q):
        pltpu.async_copy(y_hbm.at[eidx_v.at[q, 0]], rows_v.at[b], gsem[b])

    def gather_wait(j, b, q):
        pltpu.make_async_copy(y_hbm.at[eidx_v.at[q, 0]], rows_v.at[b], gsem[b]).wait()

    def scatter(j, b, q):
        pltpu.sync_copy(rows_v.at[b], agg_sh.at[eidx_v.at[q, 1]], add=True)

    plsc.subcore_barrier()
    for q in range(IR):
        idx_load(q, q)
    for b in range(NBUF):
        idx_wait(b, b)
        gather(b, b, b)

    def body(i, carry):
        for u in range(IR // NBUF):
            for b in range(NBUF):
                q = u * NBUF + b
                j = i * IR + q
                gather_wait(j, b, q)
                scatter(j, b, q)

                @pl.when(j + IR < SLAB)
                def _(j=j, q=q):
                    idx_load(j + IR, q)

                @pl.when(j + NBUF < SLAB)
                def _(j=j, b=b, q=q):
                    qn = (q + NBUF) % IR
                    idx_wait(j + NBUF, qn)
                    gather(j + NBUF, b, qn)
        return carry

    lax.fori_loop(0, SLAB // IR, body, 0)

    plsc.subcore_barrier()
    pltpu.sync_copy(agg_sh.at[pl.ds(s * ROWS_PER_TILE, ROWS_PER_TILE)],
                    agg_out.at[c, pl.ds(s * ROWS_PER_TILE, ROWS_PER_TILE)])


@functools.cache
def _agg_call():
    return pl.kernel(
        _agg_body,
        out_type=jax.ShapeDtypeStruct((NC, NP, H), jnp.float32),
        mesh=_mesh(),
        scratch_types=[
            pltpu.VMEM((IR, 2, CHUNK), jnp.int32),
            pltpu.VMEM((NBUF, CHUNK, H), jnp.float32),
            pltpu.VMEM_SHARED((NP, H), jnp.float32),
        ] + [pltpu.SemaphoreType.DMA] * (NBUF + IR),
    )


BN = 2000


def _lstm_body(x_hbm, wih_ref, whh_ref, bih_ref, bhh_ref, h_out, x_v, sem):
    i0 = pl.program_id(0)
    cp = pltpu.make_async_copy(x_hbm.at[pl.ds(i0 * BN, BN)], x_v, sem)
    cp.start()
    b = bih_ref[...] + bhh_ref[...]
    h = jnp.zeros((BN, H), jnp.float32)
    c = jnp.zeros((BN, H), jnp.float32)
    cp.wait()
    x_ref = x_v
    for t in range(T):
        xt = x_ref[:, t * D:(t + 1) * D].astype(jnp.bfloat16)
        gates = (jnp.dot(xt, wih_ref[...], preferred_element_type=jnp.float32)
                 + jnp.dot(h.astype(jnp.bfloat16), whh_ref[...],
                           preferred_element_type=jnp.float32)
                 + b)
        i = jax.nn.sigmoid(gates[:, 0:H])
        f = jax.nn.sigmoid(gates[:, H:2 * H])
        g = jnp.tanh(gates[:, 2 * H:3 * H])
        o = jax.nn.sigmoid(gates[:, 3 * H:4 * H])
        c = f * c + i * g
        h = o * jnp.tanh(c)
    h_out[...] = h


def _lstm_call(x, wihT, whhT, bih, bhh):
    return pl.pallas_call(
        _lstm_body,
        grid=(N // BN,),
        in_specs=[
            pl.BlockSpec(memory_space=pl.ANY),
            pl.BlockSpec((D, 4 * H), lambda i: (0, 0)),
            pl.BlockSpec((H, 4 * H), lambda i: (0, 0)),
            pl.BlockSpec((1, 4 * H), lambda i: (0, 0)),
            pl.BlockSpec((1, 4 * H), lambda i: (0, 0)),
        ],
        out_specs=pl.BlockSpec((BN, H), lambda i: (i, 0)),
        out_shape=jax.ShapeDtypeStruct((N, H), jnp.float32),
        scratch_shapes=[pltpu.VMEM((BN, T * D), jnp.float32),
                        pltpu.SemaphoreType.DMA],
    )(x, wihT, whhT, bih, bhh)


def _y_body(h_ref, wg_ref, degp_ref, y_out):
    deg = degp_ref[:, 0] + degp_ref[:, 1] + 1.0
    dinv = lax.rsqrt(deg)
    xw = jnp.dot(h_ref[...], wg_ref[...], preferred_element_type=jnp.float32)
    y_out[...] = xw * dinv[:, None]


def _y_call(h, wgT, degpT):
    return pl.pallas_call(
        _y_body,
        grid=(N // BN,),
        in_specs=[
            pl.BlockSpec((BN, H), lambda i: (i, 0)),
            pl.BlockSpec((H, H), lambda i: (0, 0)),
            pl.BlockSpec((BN, NC), lambda i: (i, 0)),
        ],
        out_specs=pl.BlockSpec((BN, H), lambda i: (i, 0)),
        out_shape=jax.ShapeDtypeStruct((N, H), jnp.float32),
    )(h, wgT, degpT)


def _fin_body(p_ref, y_ref, degp_ref, bg_ref, wf_ref, bf_ref, out_ref):
    deg = degp_ref[:, 0] + degp_ref[:, 1] + 1.0
    dinv = lax.rsqrt(deg)
    aggr = p_ref[0] + p_ref[1] + y_ref[...]
    agg = aggr * dinv[:, None] + bg_ref[...]
    r = jnp.maximum(agg, 0.0)
    out_ref[...] = jnp.dot(r, wf_ref[...], preferred_element_type=jnp.float32) + bf_ref[...]


def _fin_call(p, y, degpT, bg, wfT, bf):
    return pl.pallas_call(
        _fin_body,
        grid=(N // BN,),
        in_specs=[
            pl.BlockSpec((NC, BN, H), lambda i: (0, i, 0)),
            pl.BlockSpec((BN, H), lambda i: (i, 0)),
            pl.BlockSpec((BN, NC), lambda i: (i, 0)),
            pl.BlockSpec((1, H), lambda i: (0, 0)),
            pl.BlockSpec((H, O), lambda i: (0, 0)),
            pl.BlockSpec((1, O), lambda i: (0, 0)),
        ],
        out_specs=pl.BlockSpec((BN, O), lambda i: (i, 0)),
        out_shape=jax.ShapeDtypeStruct((N, O), jnp.float32),
    )(p, y, degpT, bg, wfT, bf)


def kernel(x, edge_index, W_ih, W_hh, b_ih, b_hh, W_gcn, b_gcn, W_fc, b_fc):
    it = jnp.arange(EPAD, dtype=jnp.int32)
    pad = jnp.stack([it % N, N + (it % (NP - N))])
    eidx = (jnp.concatenate([edge_index, pad], axis=1)
            .reshape(2, PCHUNKS, CHUNK).transpose(1, 0, 2))
    zrow = jnp.zeros((ROWS_PER_TILE,), jnp.float32)
    zblk = jnp.zeros((ROWS_PER_TILE, H), jnp.float32)

    degp = _deg_call()(eidx, zrow)
    degpT = degp.T
    h = _lstm_call(x.reshape(N, T * D), W_ih.T.astype(jnp.bfloat16),
                   W_hh.T.astype(jnp.bfloat16), b_ih[None, :], b_hh[None, :])
    y = _y_call(h, W_gcn.T, degpT)
    aggp = _agg_call()(eidx, y, zblk)
    out = _fin_call(aggp, y, degpT, b_gcn[None, :], W_fc.T, b_fc[None, :])
    return out


# speedup vs baseline: 1.0584x; 1.0584x over previous
import functools

import jax
import jax.numpy as jnp
from jax import lax
from jax.experimental import pallas as pl
from jax.experimental.pallas import tpu as pltpu
from jax.experimental.pallas import tpu_sc as plsc

N = 10000
E = 320000
T = 5
D = 128
H = 128
O = 128

NC = 2
NS = 16
NW = NC * NS
NP = 10240
ROWS_PER_TILE = NP // NS
CHUNK = 120
PCHUNKS = 2688
SLAB = PCHUNKS // NW
NBUF = 3
IR = 2 * NBUF
EPAD = PCHUNKS * CHUNK - E

@functools.cache
def _mesh():
    return plsc.VectorSubcoreMesh(core_axis_name="c", subcore_axis_name="s",
                                  num_cores=NC, num_subcores=NS)


def _deg_body(eidx_hbm, zeros_hbm, deg_out, eidx_v, ones_v, deg_sh, ssem):
    c = lax.axis_index("c")
    s = lax.axis_index("s")
    wid = s * NC + c
    c0 = wid * SLAB
    pltpu.sync_copy(zeros_hbm, deg_sh.at[pl.ds(s * ROWS_PER_TILE, ROWS_PER_TILE)])
    pltpu.sync_copy(eidx_hbm.at[pl.ds(c0, SLAB)], eidx_v)
    for i in range(128 // 16):
        ones_v[pl.ds(i * 16, 16)] = jnp.full((16,), 1.0, jnp.float32)
    plsc.subcore_barrier()

    def fire(j, carry):
        pltpu.async_copy(ones_v.at[pl.ds(0, CHUNK)],
                         deg_sh.at[eidx_v.at[j, 1]], ssem, add=True)
        return carry

    lax.fori_loop(0, SLAB, fire, 0)

    def drain(j, carry):
        pltpu.make_async_copy(ones_v.at[pl.ds(0, CHUNK)],
                              deg_sh.at[eidx_v.at[0, 1]], ssem).wait()
        return carry

    lax.fori_loop(0, SLAB, drain, 0)

    plsc.subcore_barrier()
    pltpu.sync_copy(deg_sh.at[pl.ds(s * ROWS_PER_TILE, ROWS_PER_TILE)],
                    deg_out.at[c, pl.ds(s * ROWS_PER_TILE, ROWS_PER_TILE)])


@functools.cache
def _deg_call():
    return pl.kernel(
        _deg_body,
        out_type=jax.ShapeDtypeStruct((NC, NP), jnp.float32),
        mesh=_mesh(),
        scratch_types=[
            pltpu.VMEM((SLAB, 2, CHUNK), jnp.int32),
            pltpu.VMEM((128,), jnp.float32),
            pltpu.VMEM_SHARED((NP,), jnp.float32),
            pltpu.SemaphoreType.DMA,
        ],
    )


def _agg_body(eidx_hbm, y_hbm, zeros_hbm, agg_out,
              eidx_v, rows_v, agg_sh, *sems):
    gsem = sems[:NBUF]
    isem = sems[NBUF:]
    c = lax.axis_index("c")
    s = lax.axis_index("s")
    wid = s * NC + c
    c0 = wid * SLAB
    pltpu.sync_copy(zeros_hbm, agg_sh.at[pl.ds(s * ROWS_PER_TILE, ROWS_PER_TILE)])

    def idx_load(j, q):
        pltpu.async_copy(eidx_hbm.at[c0 + j], eidx_v.at[q], isem[q])

    def idx_wait(j, q):
        pltpu.make_async_copy(eidx_hbm.at[c0 + j], eidx_v.at[q], isem[q]).wait()

    def gather(j, b, q):
        pltpu.async_copy(y_hbm.at[eidx_v.at[q, 0]], rows_v.at[b], gsem[b])

    def gather_wait(j, b, q):
        pltpu.make_async_copy(y_hbm.at[eidx_v.at[q, 0]], rows_v.at[b], gsem[b]).wait()

    def scatter(j, b, q):
        pltpu.sync_copy(rows_v.at[b], agg_sh.at[eidx_v.at[q, 1]], add=True)

    plsc.subcore_barrier()
    for q in range(IR):
        idx_load(q, q)
    for b in range(NBUF):
        idx_wait(b, b)
        gather(b, b, b)

    def body(i, carry):
        for u in range(IR // NBUF):
            for b in range(NBUF):
                q = u * NBUF + b
                j = i * IR + q
                gather_wait(j, b, q)
                scatter(j, b, q)

                @pl.when(j + IR < SLAB)
                def _(j=j, q=q):
                    idx_load(j + IR, q)

                @pl.when(j + NBUF < SLAB)
                def _(j=j, b=b, q=q):
                    qn = (q + NBUF) % IR
                    idx_wait(j + NBUF, qn)
                    gather(j + NBUF, b, qn)
        return carry

    lax.fori_loop(0, SLAB // IR, body, 0)

    plsc.subcore_barrier()
    pltpu.sync_copy(agg_sh.at[pl.ds(s * ROWS_PER_TILE, ROWS_PER_TILE)],
                    agg_out.at[c, pl.ds(s * ROWS_PER_TILE, ROWS_PER_TILE)])


@functools.cache
def _agg_call():
    return pl.kernel(
        _agg_body,
        out_type=jax.ShapeDtypeStruct((NC, NP, H), jnp.float32),
        mesh=_mesh(),
        scratch_types=[
            pltpu.VMEM((IR, 2, CHUNK), jnp.int32),
            pltpu.VMEM((NBUF, CHUNK, H), jnp.float32),
            pltpu.VMEM_SHARED((NP, H), jnp.float32),
        ] + [pltpu.SemaphoreType.DMA] * (NBUF + IR),
    )


BN = 2000


def _lstm_body(x_ref, wih_ref, whh_ref, bih_ref, bhh_ref, h_out):
    b = bih_ref[...] + bhh_ref[...]
    h = jnp.zeros((BN, H), jnp.float32)
    c = jnp.zeros((BN, H), jnp.float32)
    for t in range(T):
        xt = x_ref[:, t * D:(t + 1) * D].astype(jnp.bfloat16)
        gates = (jnp.dot(xt, wih_ref[...], preferred_element_type=jnp.float32)
                 + jnp.dot(h.astype(jnp.bfloat16), whh_ref[...],
                           preferred_element_type=jnp.float32)
                 + b)
        i = jax.nn.sigmoid(gates[:, 0:H])
        f = jax.nn.sigmoid(gates[:, H:2 * H])
        g = jnp.tanh(gates[:, 2 * H:3 * H])
        o = jax.nn.sigmoid(gates[:, 3 * H:4 * H])
        c = f * c + i * g
        h = o * jnp.tanh(c)
    h_out[...] = h


def _lstm_call(x, wihT, whhT, bih, bhh):
    return pl.pallas_call(
        _lstm_body,
        grid=(N // BN,),
        in_specs=[
            pl.BlockSpec((BN, T * D), lambda i: (i, 0)),
            pl.BlockSpec((D, 4 * H), lambda i: (0, 0)),
            pl.BlockSpec((H, 4 * H), lambda i: (0, 0)),
            pl.BlockSpec((1, 4 * H), lambda i: (0, 0)),
            pl.BlockSpec((1, 4 * H), lambda i: (0, 0)),
        ],
        out_specs=pl.BlockSpec((BN, H), lambda i: (i, 0)),
        out_shape=jax.ShapeDtypeStruct((N, H), jnp.float32),
    )(x, wihT, whhT, bih, bhh)


def _y_body(h_ref, wg_ref, degp_ref, y_out):
    deg = degp_ref[:, 0] + degp_ref[:, 1] + 1.0
    dinv = lax.rsqrt(deg)
    xw = jnp.dot(h_ref[...], wg_ref[...], preferred_element_type=jnp.float32)
    y_out[...] = xw * dinv[:, None]


def _y_call(h, wgT, degpT):
    return pl.pallas_call(
        _y_body,
        grid=(N // BN,),
        in_specs=[
            pl.BlockSpec((BN, H), lambda i: (i, 0)),
            pl.BlockSpec((H, H), lambda i: (0, 0)),
            pl.BlockSpec((BN, NC), lambda i: (i, 0)),
        ],
        out_specs=pl.BlockSpec((BN, H), lambda i: (i, 0)),
        out_shape=jax.ShapeDtypeStruct((N, H), jnp.float32),
    )(h, wgT, degpT)


def _fin_body(p_ref, y_ref, degp_ref, bg_ref, wf_ref, bf_ref, out_ref):
    deg = degp_ref[:, 0] + degp_ref[:, 1] + 1.0
    dinv = lax.rsqrt(deg)
    aggr = p_ref[0] + p_ref[1] + y_ref[...]
    agg = aggr * dinv[:, None] + bg_ref[...]
    r = jnp.maximum(agg, 0.0)
    out_ref[...] = jnp.dot(r, wf_ref[...], preferred_element_type=jnp.float32) + bf_ref[...]


def _fin_call(p, y, degpT, bg, wfT, bf):
    return pl.pallas_call(
        _fin_body,
        grid=(N // BN,),
        in_specs=[
            pl.BlockSpec((NC, BN, H), lambda i: (0, i, 0)),
            pl.BlockSpec((BN, H), lambda i: (i, 0)),
            pl.BlockSpec((BN, NC), lambda i: (i, 0)),
            pl.BlockSpec((1, H), lambda i: (0, 0)),
            pl.BlockSpec((H, O), lambda i: (0, 0)),
            pl.BlockSpec((1, O), lambda i: (0, 0)),
        ],
        out_specs=pl.BlockSpec((BN, O), lambda i: (i, 0)),
        out_shape=jax.ShapeDtypeStruct((N, O), jnp.float32),
    )(p, y, degpT, bg, wfT, bf)


def kernel(x, edge_index, W_ih, W_hh, b_ih, b_hh, W_gcn, b_gcn, W_fc, b_fc):
    it = jnp.arange(EPAD, dtype=jnp.int32)
    pad = jnp.stack([it % N, N + (it % (NP - N))])
    eidx = (jnp.concatenate([edge_index, pad], axis=1)
            .reshape(2, PCHUNKS, CHUNK).transpose(1, 0, 2))
    zrow = jnp.zeros((ROWS_PER_TILE,), jnp.float32)
    zblk = jnp.zeros((ROWS_PER_TILE, H), jnp.float32)

    degp = _deg_call()(eidx, zrow)
    degpT = degp.T
    h = _lstm_call(x.reshape(N, T * D), W_ih.T.astype(jnp.bfloat16),
                   W_hh.T.astype(jnp.bfloat16), b_ih[None, :], b_hh[None, :])
    y = _y_call(h, W_gcn.T, degpT)
    aggp = _agg_call()(eidx, y, zblk)
    out = _fin_call(aggp, y, degpT, b_gcn[None, :], W_fc.T, b_fc[None, :])
    return out
